# baseline (speedup 1.0000x reference)
import jax
import jax.numpy as jnp
from jax import lax
from jax.experimental import pallas as pl
from jax.experimental.pallas import tpu as pltpu
from jax.experimental.pallas import tpu_sc as plsc

T, N, E, D, H = 3, 10000, 320000, 128, 8
DH = D // H
HT = 8
HD = D // HT
WROW = D + 16
NC, NS = 2, 16
NW = NC * NS
EPT = E // NW
C = 80
NCH = EPT // C
RPS = N // NS
ZR = 125
RB = 1000


def _pre_body(x_ref, wcat_ref, asrc_ref, adst_ref, dtab_ref, as_ref):
    x = x_ref[0]
    hh = jnp.dot(x, wcat_ref[...], preferred_element_type=jnp.float32)
    dtab_ref[0, :, :D] = hh
    dtab_ref[0, :, D:] = jnp.dot(hh, adst_ref[...],
                                 preferred_element_type=jnp.float32)
    as_ref[0] = jnp.dot(hh, asrc_ref[...], preferred_element_type=jnp.float32)


def _tc_pre(features, Wcat, Asrc16, Adst16):
    return pl.pallas_call(
        _pre_body,
        grid=(T, N // RB),
        in_specs=[
            pl.BlockSpec((1, RB, D), lambda t, i: (t, i, 0)),
            pl.BlockSpec((D, D), lambda t, i: (0, 0)),
            pl.BlockSpec((D, 16), lambda t, i: (0, 0)),
            pl.BlockSpec((D, 16), lambda t, i: (0, 0)),
        ],
        out_specs=[
            pl.BlockSpec((1, RB, WROW), lambda t, i: (t, i, 0)),
            pl.BlockSpec((1, RB, 16), lambda t, i: (t, i, 0)),
        ],
        out_shape=[
            jax.ShapeDtypeStruct((T, N, WROW), jnp.float32),
            jax.ShapeDtypeStruct((T, N, 16), jnp.float32),
        ],
    )(features, Wcat, Asrc16, Adst16)


def _bcast_lane(vec, lane):
    idx = jnp.full((16, 1), lane, dtype=jnp.int32)
    dn = lax.GatherDimensionNumbers(
        offset_dims=(), collapsed_slice_dims=(0,), start_index_map=(0,))
    return lax.gather(vec, idx, dn, (1,),
                      mode=lax.GatherScatterMode.PROMISE_IN_BOUNDS)


_GROUPS = tuple((16 * g, 0) for g in range(C // 16))


def _sc_body(dtab, stab, pk_h, ev_h, zer_h, out,
             pk0, ev_l0, src_s0, drows0, srows0,
             pk1, ev_l1, src_s1, drows1, srows1, acc,
             sem_la0, sem_la1, sem_gd0, sem_gd1, sem_gs0, sem_gs1,
             sem_sc0, sem_sc1):
    cid = lax.axis_index("c")
    sid = lax.axis_index("s")
    wid = sid * NC + cid
    pk = (pk0, pk1)
    ev_l = (ev_l0, ev_l1)
    src_s = (src_s0, src_s1)
    drows = (drows0, drows1)
    srows = (srows0, srows1)
    sem_la = (sem_la0, sem_la1)
    sem_gd = (sem_gd0, sem_gd1)
    sem_gs = (sem_gs0, sem_gs1)
    sem_sc = (sem_sc0, sem_sc1)

    def zero_acc():
        for j in range(RPS // ZR):
            pltpu.sync_copy(zer_h, acc.at[pl.ds(sid * RPS + j * ZR, ZR)])

    def issue_lists(t, j, b):
        pltpu.async_copy(pk_h.at[t, wid, j], pk[b], sem_la[b])
        pltpu.async_copy(ev_h.at[t, wid, j], ev_l[b], sem_la[b])

    def wait_lists(b):
        pltpu.make_async_copy(pk_h.at[0, wid, 0], pk[b], sem_la[b]).wait()
        pltpu.make_async_copy(ev_h.at[0, wid, 0], ev_l[b], sem_la[b]).wait()

    def issue_gathers(t, b):
        pltpu.async_copy(dtab.at[t].at[pk[b].at[1]], drows[b], sem_gd[b])
        pltpu.async_copy(stab.at[t].at[pk[b].at[0]], srows[b], sem_gs[b])

    def wait_gathers(b):
        pltpu.make_async_copy(dtab.at[0].at[pk[b].at[1]], drows[b],
                              sem_gd[b]).wait()
        pltpu.make_async_copy(stab.at[0].at[pk[b].at[0]], srows[b],
                              sem_gs[b]).wait()

    def issue_scatter(b):
        pltpu.async_copy(drows[b], acc.at[src_s[b]], sem_sc[b], add=True)

    def wait_scatter(b):
        pltpu.make_async_copy(drows[b], acc.at[src_s[b]], sem_sc[b]).wait()

    def compute(b):
        for off, _ in _GROUPS:
            src_s[b][pl.ds(off, 16)] = pk[b][0, pl.ds(off, 16)]
        def group(g, carry):
            evv = ev_l[b][pl.ds(pl.multiple_of(g * 16, 16), 16)]
            for l in range(16):
                e = g * 16 + l
                evb = _bcast_lane(evv, l)
                sr = srows[b][e, :]
                ad = drows[b][e, D:WROW]
                lg = evb * (sr + ad)
                eev = jnp.exp(jnp.where(lg > 0.0, lg, lg * 0.2))
                for h in range(H):
                    seg = drows[b][e, pl.ds(h * DH, DH)]
                    drows[b][e, pl.ds(h * DH, DH)] = seg * _bcast_lane(eev, h)
                drows[b][e, D:WROW] = eev
            return carry

        lax.fori_loop(0, C // 16, group, 0)

    def tstep(t, carry):
        zero_acc()
        issue_lists(t, 0, 0)
        issue_lists(t, 1, 1)
        wait_lists(0)
        issue_gathers(t, 0)

        plsc.subcore_barrier()

        def dstep(j2, carry2):
            for b in (0, 1):
                j = 2 * j2 + b
                nb = 1 - b
                wait_gathers(b)
                compute(b)
                issue_scatter(b)

                @pl.when(jnp.logical_or(b == 0, j2 < NCH // 2 - 1))
                def _():
                    issue_lists(t, j + 2, b)

                @pl.when(jnp.logical_or(b == 1, j2 >= 1))
                def _():
                    wait_scatter(nb)
                wait_lists(nb)
                issue_gathers(t, nb)
            return carry2

        lax.fori_loop(0, NCH // 2, dstep, 0)

        wait_gathers(0)
        compute(0)
        issue_scatter(0)
        wait_scatter(1)
        wait_scatter(0)
        plsc.subcore_barrier()
        pltpu.sync_copy(acc.at[pl.ds(sid * RPS, RPS)],
                        out.at[t, cid, pl.ds(sid * RPS, RPS)])
        return carry

    lax.fori_loop(0, T, tstep, 0)


_sc_edge = pl.kernel(
    _sc_body,
    out_type=jax.ShapeDtypeStruct((T, NC, N, WROW), jnp.float32),
    mesh=plsc.VectorSubcoreMesh(core_axis_name="c", subcore_axis_name="s"),
    scratch_types=[
        pltpu.VMEM((2, C), jnp.int32),
        pltpu.VMEM((C,), jnp.float32),
        pltpu.VMEM((C,), jnp.int32),
        pltpu.VMEM((C, WROW), jnp.float32),
        pltpu.VMEM((C, 16), jnp.float32),
    ] * 2 + [
        pltpu.VMEM_SHARED((N, WROW), jnp.float32),
    ] + [pltpu.SemaphoreType.DMA] * 8,
    compiler_params=pltpu.CompilerParams(use_tc_tiling_on_sc=False),
)


def _post_body(p3, wq, wk, wv, wp, bsum, bwide, out_ref):
    ti = []
    for t in range(T):
        pt = p3[t, 0] + p3[t, 1]
        hp = pt[:, :D]
        rs = pt[:, D:D + H]
        rsw = jnp.dot(rs, bwide[...], preferred_element_type=jnp.float32)
        sl = hp / rsw
        s = jnp.where(sl > 0.0, sl, jnp.exp(sl) - 1.0)
        ti.append(s + wp[t])
    q = [jnp.dot(ti[t], wq[...], preferred_element_type=jnp.float32)
         for t in range(T)]
    k = [jnp.dot(ti[t], wk[...], preferred_element_type=jnp.float32)
         for t in range(T)]
    v = [jnp.dot(ti[t], wv[...], preferred_element_type=jnp.float32)
         for t in range(T)]

    def bs(xy):
        return jnp.dot(xy, bsum[...], preferred_element_type=jnp.float32) * 0.25

    def wd(p):
        return jnp.dot(p, bwide[...], preferred_element_type=jnp.float32)

    out0 = v[0]
    s10 = bs(q[1] * k[0])
    s11 = bs(q[1] * k[1])
    m1 = jnp.maximum(s10, s11)
    e10 = jnp.exp(s10 - m1)
    e11 = jnp.exp(s11 - m1)
    d1 = e10 + e11
    out1 = wd(e10 / d1) * v[0] + wd(e11 / d1) * v[1]
    s20 = bs(q[2] * k[0])
    s21 = bs(q[2] * k[1])
    s22 = bs(q[2] * k[2])
    m2 = jnp.maximum(jnp.maximum(s20, s21), s22)
    e20 = jnp.exp(s20 - m2)
    e21 = jnp.exp(s21 - m2)
    e22 = jnp.exp(s22 - m2)
    d2 = e20 + e21 + e22
    out2 = wd(e20 / d2) * v[0] + wd(e21 / d2) * v[1] + wd(e22 / d2) * v[2]
    outs = (out0, out1, out2)
    for t in range(T):
        out_ref[:, t, :] = outs[t] + ti[t]


def _tc_post(p3, Wq, Wk, Wv, Wp, bsum, bwide):
    def full(shape):
        return pl.BlockSpec(shape, lambda i: tuple(0 for _ in shape))
    return pl.pallas_call(
        _post_body,
        grid=(N // RB,),
        in_specs=[
            pl.BlockSpec((T, NC, RB, WROW), lambda i: (0, 0, i, 0)),
            full((D, D)), full((D, D)), full((D, D)),
            full((T, D)), full((D, HT)), full((HT, D)),
        ],
        out_specs=pl.BlockSpec((RB, T, D), lambda i: (i, 0, 0)),
        out_shape=jax.ShapeDtypeStruct((N, T, D), jnp.float32),
    )(p3, Wq, Wk, Wv, Wp, bsum, bwide)


def kernel(features, edge_index, edge_vals, W, a, Wq, Wk, Wv, Wp):
    f32 = jnp.float32
    Wcat = jnp.transpose(W, (1, 0, 2)).reshape(D, D)
    eye = jnp.eye(H, dtype=f32)
    a_src = a[:, 0, :DH]
    a_dst = a[:, 0, DH:]
    Asrc = (eye[:, None, :] * a_src[:, :, None]).reshape(D, H)
    Adst = (eye[:, None, :] * a_dst[:, :, None]).reshape(D, H)
    pad = jnp.zeros((D, 8), f32)
    Asrc16 = jnp.concatenate([Asrc, pad], axis=1)
    Adst16 = jnp.concatenate([Adst, pad], axis=1)

    dtab, as16 = _tc_pre(features, Wcat, Asrc16, Adst16)

    zer = jnp.zeros((ZR, WROW), f32)
    src2 = edge_index[:, 0, :].reshape(T, NW, NCH, C)
    dst2 = edge_index[:, 1, :].reshape(T, NW, NCH, C)
    ev2 = edge_vals.reshape(T, NW, NCH, C)
    pk = jnp.stack([src2, dst2], axis=3)

    p3 = _sc_edge(dtab, as16, pk, ev2, zer)

    bsum = jnp.repeat(eye, HD, axis=0)
    bwide = bsum.T
    return _tc_post(p3, Wq, Wk, Wv, Wp, bsum, bwide)


# speedup vs baseline: 1.3386x; 1.3386x over previous
import jax
import jax.numpy as jnp
from jax import lax
from jax.experimental import pallas as pl
from jax.experimental.pallas import tpu as pltpu
from jax.experimental.pallas import tpu_sc as plsc

T, N, E, D, H = 3, 10000, 320000, 128, 8
DH = D // H
HT = 8
HD = D // HT
WROW = D + 16
NC, NS = 2, 16
NW = NC * NS
EPT = E // NW
C = 40
NCH = EPT // C
RPS = N // NS
ZR = 125
RB = 1000


def _pre_body(x_ref, wcat_ref, asrc_ref, adst_ref, dtab_ref, as_ref):
    x = x_ref[0]
    hh = jnp.dot(x, wcat_ref[...], preferred_element_type=jnp.float32)
    dtab_ref[0, :, :D] = hh
    dtab_ref[0, :, D:] = jnp.dot(hh, adst_ref[...],
                                 preferred_element_type=jnp.float32)
    as_ref[0] = jnp.dot(hh, asrc_ref[...], preferred_element_type=jnp.float32)


def _tc_pre(features, Wcat, Asrc16, Adst16):
    return pl.pallas_call(
        _pre_body,
        grid=(T, N // RB),
        in_specs=[
            pl.BlockSpec((1, RB, D), lambda t, i: (t, i, 0)),
            pl.BlockSpec((D, D), lambda t, i: (0, 0)),
            pl.BlockSpec((D, 16), lambda t, i: (0, 0)),
            pl.BlockSpec((D, 16), lambda t, i: (0, 0)),
        ],
        out_specs=[
            pl.BlockSpec((1, RB, WROW), lambda t, i: (t, i, 0)),
            pl.BlockSpec((1, RB, 16), lambda t, i: (t, i, 0)),
        ],
        out_shape=[
            jax.ShapeDtypeStruct((T, N, WROW), jnp.float32),
            jax.ShapeDtypeStruct((T, N, 16), jnp.float32),
        ],
    )(features, Wcat, Asrc16, Adst16)


def _bcast_lane(vec, lane):
    idx = jnp.full((16, 1), lane, dtype=jnp.int32)
    dn = lax.GatherDimensionNumbers(
        offset_dims=(), collapsed_slice_dims=(0,), start_index_map=(0,))
    return lax.gather(vec, idx, dn, (1,),
                      mode=lax.GatherScatterMode.PROMISE_IN_BOUNDS)


_GROUPS = ((0, 0), (16, 0), (24, 8))


def _sc_body(dtab, stab, pk_h, ev_h, zer_h, out,
             pk0, ev_l0, src_s0, drows0, srows0, orows0,
             pk1, ev_l1, src_s1, drows1, srows1, orows1, acc,
             sem_la0, sem_la1, sem_gd0, sem_gd1, sem_gs0, sem_gs1,
             sem_sc0, sem_sc1):
    cid = lax.axis_index("c")
    sid = lax.axis_index("s")
    wid = sid * NC + cid
    pk = (pk0, pk1)
    ev_l = (ev_l0, ev_l1)
    src_s = (src_s0, src_s1)
    drows = (drows0, drows1)
    srows = (srows0, srows1)
    orows = (orows0, orows1)
    sem_la = (sem_la0, sem_la1)
    sem_gd = (sem_gd0, sem_gd1)
    sem_gs = (sem_gs0, sem_gs1)
    sem_sc = (sem_sc0, sem_sc1)

    def zero_acc():
        for j in range(RPS // ZR):
            pltpu.sync_copy(zer_h, acc.at[pl.ds(sid * RPS + j * ZR, ZR)])

    def issue_lists(t, j, b):
        pltpu.async_copy(pk_h.at[t, wid, j], pk[b], sem_la[b])
        pltpu.async_copy(ev_h.at[t, wid, j], ev_l[b], sem_la[b])

    def wait_lists(b):
        pltpu.make_async_copy(pk_h.at[0, wid, 0], pk[b], sem_la[b]).wait()
        pltpu.make_async_copy(ev_h.at[0, wid, 0], ev_l[b], sem_la[b]).wait()

    def issue_gathers(t, b):
        pltpu.async_copy(dtab.at[t].at[pk[b].at[1]], drows[b], sem_gd[b])
        pltpu.async_copy(stab.at[t].at[pk[b].at[0]], srows[b], sem_gs[b])

    def wait_gathers(b):
        pltpu.make_async_copy(dtab.at[0].at[pk[b].at[1]], drows[b],
                              sem_gd[b]).wait()
        pltpu.make_async_copy(stab.at[0].at[pk[b].at[0]], srows[b],
                              sem_gs[b]).wait()

    def issue_scatter(b):
        pltpu.async_copy(orows[b], acc.at[src_s[b]], sem_sc[b], add=True)

    def wait_scatter(b):
        pltpu.make_async_copy(orows[b], acc.at[src_s[b]], sem_sc[b]).wait()

    def compute(b):
        for off, _ in _GROUPS:
            src_s[b][pl.ds(off, 16)] = pk[b][0, pl.ds(off, 16)]
        for off, l0 in _GROUPS:
            evv = ev_l[b][pl.ds(off, 16)]
            for l in range(l0, 16):
                e = off + l
                evb = _bcast_lane(evv, l)
                sr = srows[b][e, :]
                ad = drows[b][e, D:WROW]
                lg = evb * (sr + ad)
                eev = jnp.exp(jnp.where(lg > 0.0, lg, lg * 0.2))
                for h in range(H):
                    seg = drows[b][e, pl.ds(h * DH, DH)]
                    orows[b][e, pl.ds(h * DH, DH)] = seg * _bcast_lane(eev, h)
                orows[b][e, D:WROW] = eev

    def tstep(t, carry):
        zero_acc()
        issue_lists(t, 0, 0)
        issue_lists(t, 1, 1)
        wait_lists(0)
        issue_gathers(t, 0)

        plsc.subcore_barrier()

        def dstep(j2, carry2):
            for b in (0, 1):
                j = 2 * j2 + b
                nb = 1 - b

                @pl.when(jnp.logical_or(b == 0, j2 < NCH // 2 - 1))
                def _():
                    wait_lists(nb)
                    issue_gathers(t, nb)

                wait_gathers(b)

                @pl.when(j2 >= 1)
                def _():
                    wait_scatter(b)
                compute(b)
                issue_scatter(b)

                @pl.when(j2 < NCH // 2 - 1)
                def _():
                    issue_lists(t, j + 2, b)
            return carry2

        lax.fori_loop(0, NCH // 2, dstep, 0)

        wait_scatter(0)
        wait_scatter(1)
        plsc.subcore_barrier()
        pltpu.sync_copy(acc.at[pl.ds(sid * RPS, RPS)],
                        out.at[t, cid, pl.ds(sid * RPS, RPS)])
        return carry

    lax.fori_loop(0, T, tstep, 0)


_sc_edge = pl.kernel(
    _sc_body,
    out_type=jax.ShapeDtypeStruct((T, NC, N, WROW), jnp.float32),
    mesh=plsc.VectorSubcoreMesh(core_axis_name="c", subcore_axis_name="s"),
    scratch_types=[
        pltpu.VMEM((2, C), jnp.int32),
        pltpu.VMEM((C,), jnp.float32),
        pltpu.VMEM((C,), jnp.int32),
        pltpu.VMEM((C, WROW), jnp.float32),
        pltpu.VMEM((C, 16), jnp.float32),
        pltpu.VMEM((C, WROW), jnp.float32),
    ] * 2 + [
        pltpu.VMEM_SHARED((N, WROW), jnp.float32),
    ] + [pltpu.SemaphoreType.DMA] * 8,
    compiler_params=pltpu.CompilerParams(use_tc_tiling_on_sc=False),
)


def _post_body(p3, wq, wk, wv, wp, bsum, bwide, out_ref):
    ti = []
    for t in range(T):
        pt = p3[t, 0] + p3[t, 1]
        hp = pt[:, :D]
        rs = pt[:, D:D + H]
        rsw = jnp.dot(rs, bwide[...], preferred_element_type=jnp.float32)
        sl = hp / rsw
        s = jnp.where(sl > 0.0, sl, jnp.exp(sl) - 1.0)
        ti.append(s + wp[t])
    q = [jnp.dot(ti[t], wq[...], preferred_element_type=jnp.float32)
         for t in range(T)]
    k = [jnp.dot(ti[t], wk[...], preferred_element_type=jnp.float32)
         for t in range(T)]
    v = [jnp.dot(ti[t], wv[...], preferred_element_type=jnp.float32)
         for t in range(T)]

    def bs(xy):
        return jnp.dot(xy, bsum[...], preferred_element_type=jnp.float32) * 0.25

    def wd(p):
        return jnp.dot(p, bwide[...], preferred_element_type=jnp.float32)

    out0 = v[0]
    s10 = bs(q[1] * k[0])
    s11 = bs(q[1] * k[1])
    m1 = jnp.maximum(s10, s11)
    e10 = jnp.exp(s10 - m1)
    e11 = jnp.exp(s11 - m1)
    d1 = e10 + e11
    out1 = wd(e10 / d1) * v[0] + wd(e11 / d1) * v[1]
    s20 = bs(q[2] * k[0])
    s21 = bs(q[2] * k[1])
    s22 = bs(q[2] * k[2])
    m2 = jnp.maximum(jnp.maximum(s20, s21), s22)
    e20 = jnp.exp(s20 - m2)
    e21 = jnp.exp(s21 - m2)
    e22 = jnp.exp(s22 - m2)
    d2 = e20 + e21 + e22
    out2 = wd(e20 / d2) * v[0] + wd(e21 / d2) * v[1] + wd(e22 / d2) * v[2]
    outs = (out0, out1, out2)
    for t in range(T):
        out_ref[:, t, :] = outs[t] + ti[t]


def _tc_post(p3, Wq, Wk, Wv, Wp, bsum, bwide):
    def full(shape):
        return pl.BlockSpec(shape, lambda i: tuple(0 for _ in shape))
    return pl.pallas_call(
        _post_body,
        grid=(N // RB,),
        in_specs=[
            pl.BlockSpec((T, NC, RB, WROW), lambda i: (0, 0, i, 0)),
            full((D, D)), full((D, D)), full((D, D)),
            full((T, D)), full((D, HT)), full((HT, D)),
        ],
        out_specs=pl.BlockSpec((RB, T, D), lambda i: (i, 0, 0)),
        out_shape=jax.ShapeDtypeStruct((N, T, D), jnp.float32),
    )(p3, Wq, Wk, Wv, Wp, bsum, bwide)


def kernel(features, edge_index, edge_vals, W, a, Wq, Wk, Wv, Wp):
    f32 = jnp.float32
    Wcat = jnp.transpose(W, (1, 0, 2)).reshape(D, D)
    eye = jnp.eye(H, dtype=f32)
    a_src = a[:, 0, :DH]
    a_dst = a[:, 0, DH:]
    Asrc = (eye[:, None, :] * a_src[:, :, None]).reshape(D, H)
    Adst = (eye[:, None, :] * a_dst[:, :, None]).reshape(D, H)
    pad = jnp.zeros((D, 8), f32)
    Asrc16 = jnp.concatenate([Asrc, pad], axis=1)
    Adst16 = jnp.concatenate([Adst, pad], axis=1)

    dtab, as16 = _tc_pre(features, Wcat, Asrc16, Adst16)

    zer = jnp.zeros((ZR, WROW), f32)
    src2 = edge_index[:, 0, :].reshape(T, NW, NCH, C)
    dst2 = edge_index[:, 1, :].reshape(T, NW, NCH, C)
    ev2 = edge_vals.reshape(T, NW, NCH, C)
    pk = jnp.stack([src2, dst2], axis=3)

    p3 = _sc_edge(dtab, as16, pk, ev2, zer)

    bsum = jnp.repeat(eye, HD, axis=0)
    bwide = bsum.T
    return _tc_post(p3, Wq, Wk, Wv, Wp, bsum, bwide)


# speedup vs baseline: 1.6354x; 1.2218x over previous
import jax
import jax.numpy as jnp
from jax import lax
from jax.experimental import pallas as pl
from jax.experimental.pallas import tpu as pltpu
from jax.experimental.pallas import tpu_sc as plsc

T, N, E, D, H = 3, 10000, 320000, 128, 8
DH = D // H
HT = 8
HD = D // HT
WROW = D + 16
NC, NS = 2, 16
NW = NC * NS
EPT = E // NW
C = 40
NCH = EPT // C
RPS = N // NS
ZR = 125
RB = 1000


def _pre_body(x_ref, wcat_ref, asrc_ref, adst_ref, dtab_ref, as_ref):
    x = x_ref[0]
    hh = jnp.dot(x, wcat_ref[...], preferred_element_type=jnp.float32)
    dtab_ref[0, :, :D] = hh
    dtab_ref[0, :, D:] = jnp.dot(hh, adst_ref[...],
                                 preferred_element_type=jnp.float32)
    as_ref[0] = jnp.dot(hh, asrc_ref[...], preferred_element_type=jnp.float32)


def _tc_pre(features, Wcat, Asrc16, Adst16):
    return pl.pallas_call(
        _pre_body,
        grid=(T, N // RB),
        in_specs=[
            pl.BlockSpec((1, RB, D), lambda t, i: (t, i, 0)),
            pl.BlockSpec((D, D), lambda t, i: (0, 0)),
            pl.BlockSpec((D, 16), lambda t, i: (0, 0)),
            pl.BlockSpec((D, 16), lambda t, i: (0, 0)),
        ],
        out_specs=[
            pl.BlockSpec((1, RB, WROW), lambda t, i: (t, i, 0)),
            pl.BlockSpec((1, RB, 16), lambda t, i: (t, i, 0)),
        ],
        out_shape=[
            jax.ShapeDtypeStruct((T, N, WROW), jnp.float32),
            jax.ShapeDtypeStruct((T, N, 16), jnp.float32),
        ],
    )(features, Wcat, Asrc16, Adst16)


def _bcast_lane(vec, lane):
    idx = jnp.full((16, 1), lane, dtype=jnp.int32)
    dn = lax.GatherDimensionNumbers(
        offset_dims=(), collapsed_slice_dims=(0,), start_index_map=(0,))
    return lax.gather(vec, idx, dn, (1,),
                      mode=lax.GatherScatterMode.PROMISE_IN_BOUNDS)


_GROUPS = ((0, 0), (16, 0), (24, 8))


def _sc_body(dtab, stab, ei_h, ev_h, zer_h, out,
             src_g0, dst_l0, ev_l0, src_s0, drows0, srows0, orows0,
             src_g1, dst_l1, ev_l1, src_s1, drows1, srows1, orows1, acc,
             sem_la0, sem_la1, sem_gd0, sem_gd1, sem_gs0, sem_gs1,
             sem_sc0, sem_sc1):
    cid = lax.axis_index("c")
    sid = lax.axis_index("s")
    wid = sid * NC + cid
    src_g = (src_g0, src_g1)
    dst_l = (dst_l0, dst_l1)
    ev_l = (ev_l0, ev_l1)
    src_s = (src_s0, src_s1)
    drows = (drows0, drows1)
    srows = (srows0, srows1)
    orows = (orows0, orows1)
    sem_la = (sem_la0, sem_la1)
    sem_gd = (sem_gd0, sem_gd1)
    sem_gs = (sem_gs0, sem_gs1)
    sem_sc = (sem_sc0, sem_sc1)

    def zero_acc():
        for j in range(RPS // ZR):
            pltpu.sync_copy(zer_h, acc.at[pl.ds(sid * RPS + j * ZR, ZR)])

    ebase = wid * EPT

    def issue_lists(t, j, b):
        off = ebase + j * C
        pltpu.async_copy(ei_h.at[t, 0, pl.ds(off, C)], src_g[b], sem_la[b])
        pltpu.async_copy(ei_h.at[t, 1, pl.ds(off, C)], dst_l[b], sem_la[b])
        pltpu.async_copy(ev_h.at[t, pl.ds(off, C)], ev_l[b], sem_la[b])

    def wait_lists(b):
        pltpu.make_async_copy(ei_h.at[0, 0, pl.ds(0, C)], src_g[b],
                              sem_la[b]).wait()
        pltpu.make_async_copy(ei_h.at[0, 1, pl.ds(0, C)], dst_l[b],
                              sem_la[b]).wait()
        pltpu.make_async_copy(ev_h.at[0, pl.ds(0, C)], ev_l[b],
                              sem_la[b]).wait()

    def issue_gathers(t, b):
        pltpu.async_copy(dtab.at[t].at[dst_l[b]], drows[b], sem_gd[b])
        pltpu.async_copy(stab.at[t].at[src_g[b]], srows[b], sem_gs[b])

    def wait_gathers(b):
        pltpu.make_async_copy(dtab.at[0].at[dst_l[b]], drows[b],
                              sem_gd[b]).wait()
        pltpu.make_async_copy(stab.at[0].at[src_g[b]], srows[b],
                              sem_gs[b]).wait()

    def issue_scatter(b):
        pltpu.async_copy(orows[b], acc.at[src_s[b]], sem_sc[b], add=True)

    def wait_scatter(b):
        pltpu.make_async_copy(orows[b], acc.at[src_s[b]], sem_sc[b]).wait()

    def compute(b):
        for off, _ in _GROUPS:
            src_s[b][pl.ds(off, 16)] = src_g[b][pl.ds(off, 16)]
        for off, l0 in _GROUPS:
            evv = ev_l[b][pl.ds(off, 16)]
            for l in range(l0, 16):
                e = off + l
                evb = _bcast_lane(evv, l)
                sr = srows[b][e, :]
                ad = drows[b][e, D:WROW]
                lg = evb * (sr + ad)
                eev = jnp.exp(jnp.where(lg > 0.0, lg, lg * 0.2))
                for h in range(H):
                    seg = drows[b][e, pl.ds(h * DH, DH)]
                    orows[b][e, pl.ds(h * DH, DH)] = seg * _bcast_lane(eev, h)
                orows[b][e, D:WROW] = eev

    def tstep(t, carry):
        zero_acc()
        issue_lists(t, 0, 0)
        issue_lists(t, 1, 1)
        wait_lists(0)
        issue_gathers(t, 0)

        plsc.subcore_barrier()

        def dstep(j2, carry2):
            for b in (0, 1):
                j = 2 * j2 + b
                nb = 1 - b

                @pl.when(jnp.logical_or(b == 0, j2 < NCH // 2 - 1))
                def _():
                    wait_lists(nb)
                    issue_gathers(t, nb)

                wait_gathers(b)

                @pl.when(j2 >= 1)
                def _():
                    wait_scatter(b)
                compute(b)
                issue_scatter(b)

                @pl.when(j2 < NCH // 2 - 1)
                def _():
                    issue_lists(t, j + 2, b)
            return carry2

        lax.fori_loop(0, NCH // 2, dstep, 0)

        wait_scatter(0)
        wait_scatter(1)
        plsc.subcore_barrier()
        pltpu.sync_copy(acc.at[pl.ds(sid * RPS, RPS)],
                        out.at[t, cid, pl.ds(sid * RPS, RPS)])
        return carry

    lax.fori_loop(0, T, tstep, 0)


_sc_edge = pl.kernel(
    _sc_body,
    out_type=jax.ShapeDtypeStruct((T, NC, N, WROW), jnp.float32),
    mesh=plsc.VectorSubcoreMesh(core_axis_name="c", subcore_axis_name="s"),
    scratch_types=[
        pltpu.VMEM((C,), jnp.int32),
        pltpu.VMEM((C,), jnp.int32),
        pltpu.VMEM((C,), jnp.float32),
        pltpu.VMEM((C,), jnp.int32),
        pltpu.VMEM((C, WROW), jnp.float32),
        pltpu.VMEM((C, 16), jnp.float32),
        pltpu.VMEM((C, WROW), jnp.float32),
    ] * 2 + [
        pltpu.VMEM_SHARED((N, WROW), jnp.float32),
    ] + [pltpu.SemaphoreType.DMA] * 8,
    compiler_params=pltpu.CompilerParams(use_tc_tiling_on_sc=False),
)


def _post_body(p3, wq, wk, wv, wp, bsum, bwide, out_ref):
    ti = []
    for t in range(T):
        pt = p3[t, 0] + p3[t, 1]
        hp = pt[:, :D]
        rs = pt[:, D:D + H]
        rsw = jnp.dot(rs, bwide[...], preferred_element_type=jnp.float32)
        sl = hp / rsw
        s = jnp.where(sl > 0.0, sl, jnp.exp(sl) - 1.0)
        ti.append(s + wp[t])
    q = [jnp.dot(ti[t], wq[...], preferred_element_type=jnp.float32)
         for t in range(T)]
    k = [jnp.dot(ti[t], wk[...], preferred_element_type=jnp.float32)
         for t in range(T)]
    v = [jnp.dot(ti[t], wv[...], preferred_element_type=jnp.float32)
         for t in range(T)]

    def bs(xy):
        return jnp.dot(xy, bsum[...], preferred_element_type=jnp.float32) * 0.25

    def wd(p):
        return jnp.dot(p, bwide[...], preferred_element_type=jnp.float32)

    out0 = v[0]
    s10 = bs(q[1] * k[0])
    s11 = bs(q[1] * k[1])
    m1 = jnp.maximum(s10, s11)
    e10 = jnp.exp(s10 - m1)
    e11 = jnp.exp(s11 - m1)
    d1 = e10 + e11
    out1 = wd(e10 / d1) * v[0] + wd(e11 / d1) * v[1]
    s20 = bs(q[2] * k[0])
    s21 = bs(q[2] * k[1])
    s22 = bs(q[2] * k[2])
    m2 = jnp.maximum(jnp.maximum(s20, s21), s22)
    e20 = jnp.exp(s20 - m2)
    e21 = jnp.exp(s21 - m2)
    e22 = jnp.exp(s22 - m2)
    d2 = e20 + e21 + e22
    out2 = wd(e20 / d2) * v[0] + wd(e21 / d2) * v[1] + wd(e22 / d2) * v[2]
    outs = (out0, out1, out2)
    for t in range(T):
        out_ref[:, t, :] = outs[t] + ti[t]


def _tc_post(p3, Wq, Wk, Wv, Wp, bsum, bwide):
    def full(shape):
        return pl.BlockSpec(shape, lambda i: tuple(0 for _ in shape))
    return pl.pallas_call(
        _post_body,
        grid=(N // RB,),
        in_specs=[
            pl.BlockSpec((T, NC, RB, WROW), lambda i: (0, 0, i, 0)),
            full((D, D)), full((D, D)), full((D, D)),
            full((T, D)), full((D, HT)), full((HT, D)),
        ],
        out_specs=pl.BlockSpec((RB, T, D), lambda i: (i, 0, 0)),
        out_shape=jax.ShapeDtypeStruct((N, T, D), jnp.float32),
    )(p3, Wq, Wk, Wv, Wp, bsum, bwide)


def kernel(features, edge_index, edge_vals, W, a, Wq, Wk, Wv, Wp):
    f32 = jnp.float32
    Wcat = jnp.transpose(W, (1, 0, 2)).reshape(D, D)
    eye = jnp.eye(H, dtype=f32)
    a_src = a[:, 0, :DH]
    a_dst = a[:, 0, DH:]
    Asrc = (eye[:, None, :] * a_src[:, :, None]).reshape(D, H)
    Adst = (eye[:, None, :] * a_dst[:, :, None]).reshape(D, H)
    pad = jnp.zeros((D, 8), f32)
    Asrc16 = jnp.concatenate([Asrc, pad], axis=1)
    Adst16 = jnp.concatenate([Adst, pad], axis=1)

    dtab, as16 = _tc_pre(features, Wcat, Asrc16, Adst16)

    zer = jnp.zeros((ZR, WROW), f32)
    p3 = _sc_edge(dtab, as16, edge_index, edge_vals, zer)

    bsum = jnp.repeat(eye, HD, axis=0)
    bwide = bsum.T
    return _tc_post(p3, Wq, Wk, Wv, Wp, bsum, bwide)
